# Initial kernel scaffold; baseline (speedup 1.0000x reference)
import jax
import jax.numpy as jnp
from jax.experimental import pallas as pl


def kernel(features, batch, edge_index, W, b, gamma, beta):
    raise NotImplementedError("write your pallas kernel here")


# speedup vs baseline: 4.7663x; 4.7663x over previous
import functools
import math

import jax
import jax.numpy as jnp
from jax import lax
from jax.experimental import pallas as pl
from jax.experimental.pallas import tpu as pltpu
from jax.experimental.pallas import tpu_sc as plsc

N_NODES = 10000
D = 128
N_EDGES = 320000
INV_BN = 1.0 / math.sqrt(1.0 + 1e-5)

NC = 2
NS = 16

EDGES_PER_TILE = N_EDGES // (NC * NS)
EC = 80
N_PAD = 10240
ROWS_PER_TILE = N_PAD // NS
ZR = 128

BPAD = 10240
GC = 80

_sc_mesh = plsc.VectorSubcoreMesh(core_axis_name="c", subcore_axis_name="s")


def _segment_sum_sc(features, tgt, src):
    @functools.partial(
        pl.kernel,
        out_type=jax.ShapeDtypeStruct((NC, N_PAD, D), jnp.float32),
        mesh=_sc_mesh,
        scratch_types=[
            pltpu.VMEM_SHARED((N_PAD, D), jnp.float32),
            pltpu.VMEM((EC,), jnp.int32),
            pltpu.VMEM((EC,), jnp.int32),
            pltpu.VMEM((EC, D), jnp.float32),
            pltpu.VMEM((ZR, D), jnp.float32),
            pltpu.SemaphoreType.DMA,
        ],
    )
    def k(feat_hbm, tgt_hbm, src_hbm, out_hbm, acc_sh, tgt_v, src_v, rows_v,
          zero_v, sem):
        c = lax.axis_index("c")
        s = lax.axis_index("s")

        def zrow(i, carry):
            for j in range(D // 16):
                zero_v[i, pl.ds(j * 16, 16)] = jnp.zeros((16,), jnp.float32)
            return carry

        lax.fori_loop(0, ZR, zrow, 0)
        r0 = s * ROWS_PER_TILE
        for j in range(ROWS_PER_TILE // ZR):
            pltpu.sync_copy(zero_v, acc_sh.at[pl.ds(r0 + j * ZR, ZR)])
        plsc.subcore_barrier()

        ebase = (c * NS + s) * EDGES_PER_TILE

        def step(i, carry):
            off = ebase + i * EC
            pltpu.sync_copy(tgt_hbm.at[pl.ds(off, EC)], tgt_v)
            pltpu.async_copy(feat_hbm.at[tgt_v], rows_v, sem).wait()
            pltpu.sync_copy(src_hbm.at[pl.ds(off, EC)], src_v)
            pltpu.sync_copy(rows_v, acc_sh.at[src_v], add=True)
            return carry

        lax.fori_loop(0, EDGES_PER_TILE // EC, step, 0)
        plsc.subcore_barrier()

        pltpu.sync_copy(acc_sh.at[pl.ds(r0, ROWS_PER_TILE)],
                        out_hbm.at[c, pl.ds(r0, ROWS_PER_TILE)])

    return k(features, tgt, src)


def _dense_tc(features, partials, W1, W2, b, gamma, beta):
    R = 1000

    def body(f_ref, p_ref, w1_ref, w2_ref, b_ref, g_ref, bt_ref, o_ref):
        x = f_ref[...]
        a = p_ref[0] + p_ref[1]
        dn = (((1,), (1,)), ((), ()))
        y = lax.dot_general(x, w1_ref[...], dn,
                            preferred_element_type=jnp.float32)
        y = y + lax.dot_general(a, w2_ref[...], dn,
                                preferred_element_type=jnp.float32)
        y = y + b_ref[...]
        y = jnp.maximum(y, 0.0)
        y = y * (g_ref[...] * INV_BN) + bt_ref[...]
        n = jnp.sqrt(jnp.sum(y * y, axis=1, keepdims=True))
        o_ref[...] = y / (n + 1e-6)

    return pl.pallas_call(
        body,
        grid=(N_NODES // R,),
        in_specs=[
            pl.BlockSpec((R, D), lambda i: (i, 0)),
            pl.BlockSpec((NC, R, D), lambda i: (0, i, 0)),
            pl.BlockSpec((D, D), lambda i: (0, 0)),
            pl.BlockSpec((D, D), lambda i: (0, 0)),
            pl.BlockSpec((1, D), lambda i: (0, 0)),
            pl.BlockSpec((1, D), lambda i: (0, 0)),
            pl.BlockSpec((1, D), lambda i: (0, 0)),
        ],
        out_specs=pl.BlockSpec((R, D), lambda i: (i, 0)),
        out_shape=jax.ShapeDtypeStruct((N_NODES, D), jnp.float32),
    )(features, partials, W1, W2, b, gamma, beta)


def _gather_sc(u, idx_p):
    per_w = BPAD // (NC * NS)

    @functools.partial(
        pl.kernel,
        out_type=jax.ShapeDtypeStruct((BPAD, D), jnp.float32),
        mesh=_sc_mesh,
        scratch_types=[
            pltpu.VMEM((GC,), jnp.int32),
            pltpu.VMEM((GC, D), jnp.float32),
            pltpu.SemaphoreType.DMA,
        ],
    )
    def k(u_hbm, idx_hbm, out_hbm, idx_v, rows_v, sem):
        c = lax.axis_index("c")
        s = lax.axis_index("s")
        base = (s * NC + c) * per_w

        def step(i, carry):
            off = base + i * GC
            pltpu.sync_copy(idx_hbm.at[pl.ds(off, GC)], idx_v)
            pltpu.async_copy(u_hbm.at[idx_v], rows_v, sem).wait()
            pltpu.sync_copy(rows_v, out_hbm.at[pl.ds(off, GC)])
            return carry

        lax.fori_loop(0, per_w // GC, step, 0)

    return k(u, idx_p)


def kernel(features, batch, edge_index, W, b, gamma, beta):
    src = edge_index[0].astype(jnp.int32)
    tgt = edge_index[1].astype(jnp.int32)
    bidx = batch.astype(jnp.int32)

    partials = _segment_sum_sc(features, tgt, src)
    W1 = W[:, :D]
    W2 = W[:, D:]
    u = _dense_tc(features, partials, W1, W2, b.reshape(1, D),
                  gamma.reshape(1, D), beta.reshape(1, D))
    bpad = jnp.concatenate([bidx, jnp.zeros((BPAD - N_NODES,), jnp.int32)])
    outp = _gather_sc(u, bpad)
    return outp[:N_NODES]
